# baseline (speedup 1.0000x reference)
import functools
import math

import jax
import jax.numpy as jnp
from jax.experimental import pallas as pl
from jax.experimental.pallas import tpu as pltpu

_NMS_SIZE = 15
_PYR = 4
_UP = 1
_SFL = math.sqrt(2.0)
_SMULT = 22.0
_NUM_FEATURES = 2048
_BORDERS = 15

_N_SORT = 16384
_LVL_SHIFT = 1 << 20
_BIG_IDX = _LVL_SHIFT - 1


def _level_meta():
    fp = _SFL ** 2
    levels = _PYR + _UP + 1
    tmp = 0.0
    npl = []
    for i in range(levels):
        tmp += fp ** (-(i - _UP))
        npl.append(_NUM_FEATURES * fp ** (-(i - _UP)))
    npl = [int(x / tmp) for x in npl]
    h = w = 512
    meta = []
    nh, nw = int(h * _SFL), int(w * _SFL)
    meta.append((nh, nw, int(npl[0]), float(w) / float(nw), float(h) / float(nh)))
    cur_h, cur_w = h, w
    for i in range(_PYR + 1):
        if i > 0:
            cur_h, cur_w = int(float(cur_h) / _SFL), int(float(cur_w) / _SFL)
            fx, fy = float(w) / float(cur_w), float(h) / float(cur_h)
        else:
            fx = fy = 1.0
        k = int(sum(npl[a] for a in range(i + 1 + _UP)))
        meta.append((cur_h, cur_w, k, fx, fy))
    return meta


_META = _level_meta()


def _cand_body(x_ref, val_ref, pay_ref, *, H, W, level):
    r = x_ref[...]
    det = r * r
    row = jax.lax.broadcasted_iota(jnp.int32, (H, W), 0)
    col = jax.lax.broadcasted_iota(jnp.int32, (H, W), 1)
    keep = ((row >= _BORDERS) & (row < H - _BORDERS)
            & (col >= _BORDERS) & (col < W - _BORDERS))
    det = jnp.where(keep, det, 0.0)
    m = det
    for axis, n in ((0, H), (1, W)):
        m2 = jnp.maximum(m, pltpu.roll(m, n - 1, axis))
        m4 = jnp.maximum(m2, pltpu.roll(m2, n - 2, axis))
        m8 = jnp.maximum(m4, pltpu.roll(m4, n - 4, axis))
        m15 = jnp.maximum(m8, pltpu.roll(m8, n - 7, axis))
        m = pltpu.roll(m15, 7, axis)
    det = jnp.where(det == m, det, 0.0)
    pos = det > 0.0
    flat = (row * W + col).astype(jnp.float32)
    payv = jnp.where(pos, flat, float(_BIG_IDX)) + float(level * _LVL_SHIFT)
    pad_h = (-H) % 8
    if pad_h:
        det = jnp.concatenate([det, jnp.zeros((pad_h, W), jnp.float32)], axis=0)
        payv = jnp.concatenate(
            [payv, jnp.full((pad_h, W), float(level * _LVL_SHIFT + _BIG_IDX),
                            jnp.float32)], axis=0)
    hb = det.shape[0] // 8
    detb = jnp.max(det.reshape(hb, 8, W), axis=1)
    payb = jnp.min(payv.reshape(hb, 8, W), axis=1)
    detb = detb.T
    payb = payb.T
    pad_w = (-W) % 8
    if pad_w:
        detb = jnp.concatenate([detb, jnp.zeros((pad_w, hb), jnp.float32)], axis=0)
        payb = jnp.concatenate(
            [payb, jnp.full((pad_w, hb), float(level * _LVL_SHIFT + _BIG_IDX),
                            jnp.float32)], axis=0)
    wb = detb.shape[0] // 8
    val_ref[...] = jnp.max(detb.reshape(wb, 8, hb), axis=1)
    pay_ref[...] = jnp.min(payb.reshape(wb, 8, hb), axis=1)


def _candidates(r, level):
    H, W = r.shape
    hb, wb = (H + 7) // 8, (W + 7) // 8
    return pl.pallas_call(
        functools.partial(_cand_body, H=H, W=W, level=level),
        out_shape=(jax.ShapeDtypeStruct((wb, hb), jnp.float32),
                   jax.ShapeDtypeStruct((wb, hb), jnp.float32)),
    )(r)


def _lane_xor(x, d, cbit):
    left = pltpu.roll(x, 128 - d, 1)
    right = pltpu.roll(x, d, 1)
    return jnp.where(cbit, right, left)


def _row_xor(x, d, rbit):
    left = pltpu.roll(x, 128 - d, 0)
    right = pltpu.roll(x, d, 0)
    return jnp.where(rbit, right, left)


def _bitonic(arrs, prec, i_arr, r_arr, c_arr, n_log2):
    for k in range(1, n_log2 + 1):
        for j in reversed(range(k)):
            d = 1 << j
            if d >= 128:
                dr = d >> 7
                bit = (r_arr & dr) != 0
                partners = [_row_xor(a, dr, bit) for a in arrs]
            else:
                bit = (c_arr & d) != 0
                partners = [_lane_xor(a, d, bit) for a in arrs]
            xwins = prec(arrs, partners)
            dirb = ((i_arr >> k) & 1) == 0
            lowbit = (i_arr & d) == 0
            winner_here = lowbit == dirb
            take_self = xwins == winner_here
            arrs = [jnp.where(take_self, a, pa)
                    for a, pa in zip(arrs, partners)]
    return arrs


def _select_body(val_ref, pay_ref, resp_ref, sc_ref, xx_ref, yy_ref):
    v = val_ref[...]
    p = pay_ref[...]
    r_arr = jax.lax.broadcasted_iota(jnp.int32, (128, 128), 0)
    c_arr = jax.lax.broadcasted_iota(jnp.int32, (128, 128), 1)
    i_arr = r_arr * 128 + c_arr

    def prec1(a, b):
        return (a[0] > b[0]) | ((a[0] == b[0]) & (a[1] < b[1]))

    v, p = _bitonic([v, p], prec1, i_arr, r_arr, c_arr, 14)

    tri_lane = (jax.lax.broadcasted_iota(jnp.int32, (128, 128), 0)
                <= jax.lax.broadcasted_iota(jnp.int32, (128, 128), 1))
    tri_lane = tri_lane.astype(jnp.float32)
    tri_row_strict = (jax.lax.broadcasted_iota(jnp.int32, (128, 128), 0)
                      < jax.lax.broadcasted_iota(jnp.int32, (128, 128), 1))
    tri_row_strict = tri_row_strict.astype(jnp.float32)

    def cumsum_all(flag):
        within = jax.lax.dot_general(
            flag, tri_lane, (((1,), (0,)), ((), ())),
            precision=jax.lax.Precision.HIGHEST)
        rowtot = jnp.sum(flag, axis=1, keepdims=True)
        rowpfx = jax.lax.dot_general(
            tri_row_strict, rowtot, (((0,), (0,)), ((), ())),
            precision=jax.lax.Precision.HIGHEST)
        return within + rowpfx

    lvl_f = jnp.floor(p * (1.0 / _LVL_SHIFT))
    pos = v > 0.0
    elig = jnp.zeros((128, 128), jnp.bool_)
    m_counts = []
    for L, (_H, _W, kL, _fx, _fy) in enumerate(_META):
        flag = (pos & (lvl_f == float(L))).astype(jnp.float32)
        rank = cumsum_all(flag)
        elig_L = (flag > 0) & (rank <= float(kL))
        elig = elig | elig_L
        m_counts.append(jnp.sum(elig_L.astype(jnp.float32)))

    key = i_arr.astype(jnp.float32) + jnp.where(elig, 0.0, float(_N_SORT))

    def prec2(a, b):
        return a[0] < b[0]

    key, v, p = _bitonic([key, v, p], prec2, i_arr, r_arr, c_arr, 14)

    n_elig = jnp.sum(elig.astype(jnp.float32))

    s_i = i_arr[:16, :]
    v16 = v[:16, :]
    p16 = p[:16, :]
    filled = s_i.astype(jnp.float32) < n_elig

    z = s_i.astype(jnp.float32) - n_elig
    zcum = jnp.zeros_like(z)
    zlvl = jnp.full(z.shape, float(len(_META) - 1))
    zidx = jnp.zeros_like(z)
    for L, (_H, _W, kL, _fx, _fy) in enumerate(_META):
        cap = float(kL) - m_counts[L]
        in_L = (z >= zcum) & (z < zcum + cap)
        zlvl = jnp.where(in_L, float(L), zlvl)
        zidx = jnp.where(in_L, z - zcum, zidx)
        zcum = zcum + cap

    lvl16 = jnp.where(filled, jnp.floor(p16 * (1.0 / _LVL_SHIFT)), zlvl)
    idx16 = jnp.where(filled, p16 - lvl16 * float(_LVL_SHIFT), zidx)
    resp = jnp.where(filled, jnp.maximum(v16, 0.0), 0.0)

    idx_i = idx16.astype(jnp.int32)
    xx = jnp.zeros(idx16.shape, jnp.float32)
    yy = jnp.zeros(idx16.shape, jnp.float32)
    sc = jnp.zeros(idx16.shape, jnp.float32)
    for L, (_H, Wl, _kL, fx, fy) in enumerate(_META):
        is_L = lvl16 == float(L)
        col_L = (idx_i % Wl).astype(jnp.float32)
        row_L = (idx_i // Wl).astype(jnp.float32)
        xx = jnp.where(is_L, col_L * jnp.float32(fx), xx)
        yy = jnp.where(is_L, row_L * jnp.float32(fy), yy)
        sc = jnp.where(is_L, jnp.float32(0.5 * (fx + fy) * _SMULT), sc)

    resp_ref[...] = resp
    sc_ref[...] = sc
    xx_ref[...] = xx
    yy_ref[...] = yy


def _select(val, pay):
    return pl.pallas_call(
        _select_body,
        out_shape=(jax.ShapeDtypeStruct((16, 128), jnp.float32),) * 4,
    )(val.reshape(128, 128), pay.reshape(128, 128))


def _response(img, W):
    return jax.lax.conv_general_dilated(
        img, W, (1, 1), 'SAME', dimension_numbers=('NCHW', 'OIHW', 'NCHW'))


def _pyrdown(x, factor):
    k1 = jnp.array([1., 4., 6., 4., 1.], dtype=jnp.float32) / 16.0
    kern = jnp.outer(k1, k1)[None, None]
    xp = jnp.pad(x, ((0, 0), (0, 0), (2, 2), (2, 2)), mode='reflect')
    blurred = jax.lax.conv_general_dilated(
        xp, kern, (1, 1), 'VALID', dimension_numbers=('NCHW', 'OIHW', 'NCHW'))
    h, w = x.shape[2], x.shape[3]
    nh, nw = int(float(h) / factor), int(float(w) / factor)
    return jax.image.resize(blurred, (x.shape[0], x.shape[1], nh, nw), 'bilinear')


def kernel(img, W):
    h, w = img.shape[2], img.shape[3]
    vals, pays = [], []
    nh, nw = _META[0][0], _META[0][1]
    img_up = jax.image.resize(img, (1, 1, nh, nw), 'bilinear')
    rv, rp = _candidates(_response(img_up, W)[0, 0], 0)
    vals.append(rv.reshape(-1))
    pays.append(rp.reshape(-1))
    cur = img
    for i in range(_PYR + 1):
        if i > 0:
            cur = _pyrdown(cur, _SFL)
        rv, rp = _candidates(_response(cur, W)[0, 0], i + 1)
        vals.append(rv.reshape(-1))
        pays.append(rp.reshape(-1))
    val = jnp.concatenate(vals)
    pay = jnp.concatenate(pays)
    n = val.shape[0]
    assert n <= _N_SORT, n
    val = jnp.concatenate([val, jnp.full((_N_SORT - n,), -1.0, jnp.float32)])
    pay = jnp.concatenate(
        [pay, jnp.full((_N_SORT - n,), float(8 * _LVL_SHIFT), jnp.float32)])
    resp, sc, xx, yy = _select(val, pay)
    resp = resp.reshape(1, _NUM_FEATURES)
    sc = sc.reshape(_NUM_FEATURES)
    xx = xx.reshape(_NUM_FEATURES)
    yy = yy.reshape(_NUM_FEATURES)
    zero = jnp.zeros_like(sc)
    row0 = jnp.stack([sc, zero, xx], axis=-1)
    row1 = jnp.stack([zero, sc, yy], axis=-1)
    lafs = jnp.stack([row0, row1], axis=-2)[None]
    return (lafs, resp)
